# baseline (speedup 1.0000x reference)
import jax
import jax.numpy as jnp
from jax.experimental import pallas as pl
from jax.experimental.pallas import tpu as pltpu

_BF = jnp.bfloat16
_F32 = jnp.float32


def _conv(x, w, b, stride, pad):
    out = jax.lax.conv_general_dilated(
        x, w, (stride, stride), [(pad, pad), (pad, pad)],
        dimension_numbers=('NCHW', 'OIHW', 'NCHW'))
    return out + b[None, :, None, None]


def _res(x, w1, b1, w2, b2):
    h = _conv(jax.nn.elu(x), w1, b1, 1, 1)
    h = _conv(jax.nn.elu(h), w2, b2, 1, 0)
    return x + h


def _rtne_bf16_f32(x):
    u = jax.lax.bitcast_convert_type(x, jnp.uint32)
    t = u + jnp.uint32(0x7FFF) + ((u >> 16) & jnp.uint32(1))
    return jax.lax.bitcast_convert_type(t & jnp.uint32(0xFFFF0000), _F32)


def _elu(x):
    return jnp.where(x > 0, x, jnp.exp(x) - 1.0)


_T_TAPS = {0: ((1, 1), (0, 3)), 1: ((2, 0), (1, 2))}


def _vqdec_kernel(z_ref, ebf_ref, cn_ref, eh_ref, em_ref,
                  wd1_ref, bd1_ref, wr1a_ref, br1a_ref, wr1b_ref, br1b_ref,
                  wr2a_ref, br2a_ref, wr2b_ref, br2b_ref, wt1_ref, bt1_ref,
                  o_ref, sq_ref, scr_ref):
    n = pl.program_id(0)

    z = z_ref[0].reshape(3136, 64)
    mm = jnp.dot(z.astype(_BF), ebf_ref[...], preferred_element_type=_F32)
    rown = jnp.sum(z * z, axis=1, keepdims=True)
    dist = (rown - 2.0 * mm) + cn_ref[...]
    dmin = jnp.min(dist, axis=1, keepdims=True)
    iota = jax.lax.broadcasted_iota(jnp.int32, (3136, 1024), 1)
    ind = jnp.min(jnp.where(dist == dmin, iota, jnp.int32(1 << 30)),
                  axis=1, keepdims=True)
    onehot = (iota == ind).astype(_BF)
    q = jnp.dot(onehot, eh_ref[...], preferred_element_type=_F32)
    q = q + jnp.dot(onehot, em_ref[...], preferred_element_type=_F32)
    d = q - z
    qst = z + d
    s = jnp.sum(d * d)

    @pl.when(n == 0)
    def _():
        sq_ref[...] = s.reshape(1, 1)

    @pl.when(n != 0)
    def _():
        sq_ref[...] += s.reshape(1, 1)

    scr_ref[...] = jnp.zeros((58, 58, 128), _BF)
    scr_ref[1:57, 1:57, 0:64] = qst.astype(_BF).reshape(56, 56, 64)

    def taps3(cin, w_ref):
        acc = None
        for t in range(9):
            i, j = divmod(t, 3)
            sl = scr_ref[i:i + 56, j:j + 56, 0:cin].reshape(3136, cin)
            term = jnp.dot(sl, w_ref[t], preferred_element_type=_F32)
            acc = term if acc is None else acc + term
        return acc

    h = taps3(64, wd1_ref) + bd1_ref[...]

    def res(h, wa_ref, ba_ref, wb_ref, bb_ref):
        scr_ref[1:57, 1:57, :] = _elu(h).astype(_BF).reshape(56, 56, 128)
        t = taps3(128, wa_ref) + ba_ref[...]
        t = jnp.dot(_elu(t).astype(_BF), wb_ref[...],
                    preferred_element_type=_F32) + bb_ref[...]
        return h + t

    h = res(h, wr1a_ref, br1a_ref, wr1b_ref, br1b_ref)
    h = res(h, wr2a_ref, br2a_ref, wr2b_ref, br2b_ref)

    scr_ref[1:57, 1:57, :] = _elu(h).astype(_BF).reshape(56, 56, 128)
    for rh in range(2):
        for rw in range(2):
            acc = None
            for (oh, kh) in _T_TAPS[rh]:
                for (ow, kw) in _T_TAPS[rw]:
                    sl = scr_ref[oh:oh + 56, ow:ow + 56, :].reshape(3136, 128)
                    term = jnp.dot(sl, wt1_ref[kh, kw],
                                   preferred_element_type=_F32)
                    acc = term if acc is None else acc + term
            sub = _elu(acc + bt1_ref[...])
            c0 = (rh * 2 + rw) * 64
            o_ref[0, :, :, c0:c0 + 64] = sub.reshape(56, 56, 64)


def _run_vqdec(z_nhwc, embed, p):
    embed_bf = embed.astype(_BF)
    colnorm = (embed ** 2).sum(0, keepdims=True)
    et = jnp.transpose(embed)
    eh_f = _rtne_bf16_f32(et)
    eh, em = eh_f.astype(_BF), (et - eh_f).astype(_BF)
    w3 = lambda w: jnp.transpose(w, (2, 3, 1, 0)).reshape(9, w.shape[1], w.shape[0]).astype(_BF)
    fixed = lambda n: (0, 0)
    fixed3 = lambda n: (0, 0, 0)
    args = (z_nhwc, embed_bf, colnorm, eh, em,
            w3(p['d_c1_w']), p['d_c1_b'].reshape(1, 128),
            w3(p['d_r1a_w']), p['d_r1a_b'].reshape(1, 64),
            jnp.transpose(p['d_r1b_w'][:, :, 0, 0]).astype(_BF), p['d_r1b_b'].reshape(1, 128),
            w3(p['d_r2a_w']), p['d_r2a_b'].reshape(1, 64),
            jnp.transpose(p['d_r2b_w'][:, :, 0, 0]).astype(_BF), p['d_r2b_b'].reshape(1, 128),
            jnp.transpose(p['d_t1_w'], (2, 3, 0, 1)).astype(_BF),
            p['d_t1_b'].reshape(1, 64))
    f = pl.pallas_call(
        _vqdec_kernel,
        grid=(4,),
        in_specs=[pl.BlockSpec((1, 56, 56, 64), lambda n: (n, 0, 0, 0)),
                  pl.BlockSpec((64, 1024), fixed),
                  pl.BlockSpec((1, 1024), fixed),
                  pl.BlockSpec((1024, 64), fixed),
                  pl.BlockSpec((1024, 64), fixed),
                  pl.BlockSpec((9, 64, 128), fixed3),
                  pl.BlockSpec((1, 128), fixed),
                  pl.BlockSpec((9, 128, 64), fixed3),
                  pl.BlockSpec((1, 64), fixed),
                  pl.BlockSpec((64, 128), fixed),
                  pl.BlockSpec((1, 128), fixed),
                  pl.BlockSpec((9, 128, 64), fixed3),
                  pl.BlockSpec((1, 64), fixed),
                  pl.BlockSpec((64, 128), fixed),
                  pl.BlockSpec((1, 128), fixed),
                  pl.BlockSpec((4, 4, 128, 64), lambda n: (0, 0, 0, 0)),
                  pl.BlockSpec((1, 64), fixed)],
        out_specs=[pl.BlockSpec((1, 56, 56, 256), lambda n: (n, 0, 0, 0)),
                   pl.BlockSpec((1, 1), fixed)],
        out_shape=[jax.ShapeDtypeStruct((4, 56, 56, 256), _F32),
                   jax.ShapeDtypeStruct((1, 1), _F32)],
        scratch_shapes=[pltpu.VMEM((58, 58, 128), _BF)],
    )
    return f(*args)


def _t2_kernel(x_ref, w_ref, b_ref, o_ref):
    acc = None
    for o in range(9):
        oh, ow = divmod(o, 3)
        s = x_ref[0][oh:oh + 112, ow:ow + 112, :].reshape(12544, 64)
        term = jnp.dot(s.astype(_BF), w_ref[o], preferred_element_type=_F32)
        acc = term if acc is None else acc + term
    o_ref[0] = (acc + b_ref[...]).reshape(112, 112, 12)


def _run_t2(hp, w, b):
    wt = jnp.transpose(w, (2, 3, 0, 1))
    w9 = jnp.zeros((9, 64, 12), _F32)
    for rh in range(2):
        for rw in range(2):
            for (oh, kh) in _T_TAPS[rh]:
                for (ow, kw) in _T_TAPS[rw]:
                    c0 = (rh * 2 + rw) * 3
                    w9 = w9.at[oh * 3 + ow, :, c0:c0 + 3].add(wt[kh, kw])
    b12 = jnp.tile(b, 4).reshape(1, 12)
    f = pl.pallas_call(
        _t2_kernel,
        grid=(4,),
        in_specs=[pl.BlockSpec((1, 114, 114, 64), lambda n: (n, 0, 0, 0)),
                  pl.BlockSpec((9, 64, 12), lambda n: (0, 0, 0)),
                  pl.BlockSpec((1, 12), lambda n: (0, 0))],
        out_specs=pl.BlockSpec((1, 112, 112, 12), lambda n: (n, 0, 0, 0)),
        out_shape=jax.ShapeDtypeStruct((4, 112, 112, 12), _F32),
    )
    return f(hp, w9.astype(_BF), b12)


def _d2s(y, c):
    n, h, w, _ = y.shape
    y = y.reshape(n, h, w, 2, 2, c)
    y = jnp.transpose(y, (0, 1, 3, 2, 4, 5))
    return y.reshape(n, 2 * h, 2 * w, c)


def kernel(x, e_c1_w, e_c1_b, e_c2_w, e_c2_b, e_c3_w, e_c3_b, e_r1a_w,
           e_r1a_b, e_r1b_w, e_r1b_b, e_r2a_w, e_r2a_b, e_r2b_w, e_r2b_b,
           q_w, q_b, embed, d_c1_w, d_c1_b, d_r1a_w, d_r1a_b, d_r1b_w,
           d_r1b_b, d_r2a_w, d_r2a_b, d_r2b_w, d_r2b_b, d_t1_w, d_t1_b,
           d_t2_w, d_t2_b):
    h = jax.nn.elu(_conv(x, e_c1_w, e_c1_b, 2, 1))
    h = jax.nn.elu(_conv(h, e_c2_w, e_c2_b, 2, 1))
    h = _conv(h, e_c3_w, e_c3_b, 1, 1)
    h = _res(h, e_r1a_w, e_r1a_b, e_r1b_w, e_r1b_b)
    h = _res(h, e_r2a_w, e_r2a_b, e_r2b_w, e_r2b_b)
    h = jax.nn.elu(h)
    z = _conv(h, q_w, q_b, 1, 0)
    z_nhwc = jnp.transpose(z, (0, 2, 3, 1))

    p = {'d_c1_w': d_c1_w, 'd_c1_b': d_c1_b,
         'd_r1a_w': d_r1a_w, 'd_r1a_b': d_r1a_b,
         'd_r1b_w': d_r1b_w, 'd_r1b_b': d_r1b_b,
         'd_r2a_w': d_r2a_w, 'd_r2a_b': d_r2a_b,
         'd_r2b_w': d_r2b_w, 'd_r2b_b': d_r2b_b,
         'd_t1_w': d_t1_w, 'd_t1_b': d_t1_b}
    y, sq = _run_vqdec(z_nhwc, embed, p)
    diff = 0.25 * (sq[0, 0] / (4 * 56 * 56 * 64))

    h1 = _d2s(y, 64)
    hp = jnp.zeros((4, 114, 114, 64), _F32).at[:, 1:113, 1:113, :].set(h1)
    y2 = _run_t2(hp, d_t2_w, d_t2_b)
    dec = _d2s(y2, 3)
    dec = jnp.transpose(dec, (0, 3, 1, 2))
    return dec, diff


# speedup vs baseline: 1.1482x; 1.1482x over previous
import jax
import jax.numpy as jnp
from jax.experimental import pallas as pl
from jax.experimental.pallas import tpu as pltpu

_BF = jnp.bfloat16
_F32 = jnp.float32


def _conv(x, w, b, stride, pad):
    out = jax.lax.conv_general_dilated(
        x, w, (stride, stride), [(pad, pad), (pad, pad)],
        dimension_numbers=('NCHW', 'OIHW', 'NCHW'))
    return out + b[None, :, None, None]


def _res(x, w1, b1, w2, b2):
    h = _conv(jax.nn.elu(x), w1, b1, 1, 1)
    h = _conv(jax.nn.elu(h), w2, b2, 1, 0)
    return x + h


def _rtne_bf16_f32(x):
    u = jax.lax.bitcast_convert_type(x, jnp.uint32)
    t = u + jnp.uint32(0x7FFF) + ((u >> 16) & jnp.uint32(1))
    return jax.lax.bitcast_convert_type(t & jnp.uint32(0xFFFF0000), _F32)


def _elu(x):
    return jnp.where(x > 0, x, jnp.exp(x) - 1.0)


_T_TAPS = {0: ((1, 1), (0, 3)), 1: ((2, 0), (1, 2))}
_E_MAP = {-1: (1, -1), 0: (0, 0), 1: (1, 0), 2: (0, 1)}


def _vqdec_kernel(z_ref, ebf_ref, cn_ref, eh_ref, em_ref,
                  wd1_ref, bd1_ref, wr1a_ref, br1a_ref, wr1b_ref, br1b_ref,
                  wr2a_ref, br2a_ref, wr2b_ref, br2b_ref, wt1_ref, bt1_ref,
                  w36_ref, b48_ref, o_ref, sq_ref,
                  scrq_ref, scr_ref, scrt_ref):
    n = pl.program_id(0)

    z = z_ref[0].reshape(3136, 64)
    mm = jnp.dot(z.astype(_BF), ebf_ref[...], preferred_element_type=_F32)
    rown = jnp.sum(z * z, axis=1, keepdims=True)
    dist = (rown - 2.0 * mm) + cn_ref[...]
    dmin = jnp.min(dist, axis=1, keepdims=True)
    iota = jax.lax.broadcasted_iota(jnp.int32, (3136, 1024), 1)
    ind = jnp.min(jnp.where(dist == dmin, iota, jnp.int32(1 << 30)),
                  axis=1, keepdims=True)
    onehot = (iota == ind).astype(_BF)
    q = jnp.dot(onehot, eh_ref[...], preferred_element_type=_F32)
    q = q + jnp.dot(onehot, em_ref[...], preferred_element_type=_F32)
    d = q - z
    qst = z + d
    s = jnp.sum(d * d)

    @pl.when(n == 0)
    def _():
        sq_ref[...] = s.reshape(1, 1)

    @pl.when(n != 0)
    def _():
        sq_ref[...] += s.reshape(1, 1)

    scrq_ref[...] = jnp.zeros((58, 58, 64), _F32)
    scrq_ref[1:57, 1:57, :] = qst.reshape(56, 56, 64)
    scr_ref[...] = jnp.zeros((58, 58, 128), _F32)

    def taps3(src_ref, cin, w_ref):
        acc = None
        for t in range(9):
            i, j = divmod(t, 3)
            sl = src_ref[i:i + 56, j:j + 56, :].reshape(3136, cin)
            term = jnp.dot(sl.astype(_BF), w_ref[t],
                           preferred_element_type=_F32)
            acc = term if acc is None else acc + term
        return acc

    h = taps3(scrq_ref, 64, wd1_ref) + bd1_ref[...]

    def res(h, wa_ref, ba_ref, wb_ref, bb_ref):
        scr_ref[1:57, 1:57, :] = _elu(h).reshape(56, 56, 128)
        t = taps3(scr_ref, 128, wa_ref) + ba_ref[...]
        t = jnp.dot(_elu(t).astype(_BF), wb_ref[...],
                    preferred_element_type=_F32) + bb_ref[...]
        return h + t

    h = res(h, wr1a_ref, br1a_ref, wr1b_ref, br1b_ref)
    h = res(h, wr2a_ref, br2a_ref, wr2b_ref, br2b_ref)

    scr_ref[1:57, 1:57, :] = _elu(h).reshape(56, 56, 128)
    scrt_ref[...] = jnp.zeros((58, 58, 256), _F32)
    for rh in range(2):
        for rw in range(2):
            acc = None
            for (oh, kh) in _T_TAPS[rh]:
                for (ow, kw) in _T_TAPS[rw]:
                    sl = scr_ref[oh:oh + 56, ow:ow + 56, :].reshape(3136, 128)
                    term = jnp.dot(sl.astype(_BF), wt1_ref[kh, kw],
                                   preferred_element_type=_F32)
                    acc = term if acc is None else acc + term
            sub = _elu(acc + bt1_ref[...])
            c0 = (rh * 2 + rw) * 64
            scrt_ref[1:57, 1:57, c0:c0 + 64] = sub.reshape(56, 56, 64)

    acc = None
    m = 0
    for sh in range(2):
        for dh in (-1, 0, 1):
            for sw in range(2):
                for dw in (-1, 0, 1):
                    c0 = (sh * 2 + sw) * 64
                    sl = scrt_ref[1 + dh:57 + dh, 1 + dw:57 + dw,
                                  c0:c0 + 64].reshape(3136, 64)
                    term = jnp.dot(sl.astype(_BF), w36_ref[m],
                                   preferred_element_type=_F32)
                    acc = term if acc is None else acc + term
                    m += 1
    o_ref[0] = (acc + b48_ref[...]).reshape(56, 56, 48)


def _build_w36(d_t2_w):
    wt = jnp.transpose(d_t2_w, (2, 3, 0, 1))
    order = {}
    m = 0
    for sh in range(2):
        for dh in (-1, 0, 1):
            for sw in range(2):
                for dw in (-1, 0, 1):
                    order[(sh, dh, sw, dw)] = m
                    m += 1
    w36 = jnp.zeros((36, 64, 48), _F32)
    for t in range(2):
        for rh in range(2):
            ph = 2 * t + rh
            for (oh, kh) in _T_TAPS[rh]:
                sh, dh = _E_MAP[t + oh - 1]
                for u in range(2):
                    for rw in range(2):
                        pw = 2 * u + rw
                        for (ow, kw) in _T_TAPS[rw]:
                            sw, dw = _E_MAP[u + ow - 1]
                            mi = order[(sh, dh, sw, dw)]
                            c0 = (ph * 4 + pw) * 3
                            w36 = w36.at[mi, :, c0:c0 + 3].add(wt[kh, kw])
    return w36


def _run_vqdec(z_nhwc, embed, p):
    embed_bf = embed.astype(_BF)
    colnorm = (embed ** 2).sum(0, keepdims=True)
    et = jnp.transpose(embed)
    eh_f = _rtne_bf16_f32(et)
    eh, em = eh_f.astype(_BF), (et - eh_f).astype(_BF)
    w3 = lambda w: jnp.transpose(w, (2, 3, 1, 0)).reshape(9, w.shape[1], w.shape[0]).astype(_BF)
    w36 = _build_w36(p['d_t2_w']).astype(_BF)
    b48 = jnp.tile(p['d_t2_b'], 16).reshape(1, 48)
    fixed = lambda n: (0, 0)
    fixed3 = lambda n: (0, 0, 0)
    args = (z_nhwc, embed_bf, colnorm, eh, em,
            w3(p['d_c1_w']), p['d_c1_b'].reshape(1, 128),
            w3(p['d_r1a_w']), p['d_r1a_b'].reshape(1, 64),
            jnp.transpose(p['d_r1b_w'][:, :, 0, 0]).astype(_BF), p['d_r1b_b'].reshape(1, 128),
            w3(p['d_r2a_w']), p['d_r2a_b'].reshape(1, 64),
            jnp.transpose(p['d_r2b_w'][:, :, 0, 0]).astype(_BF), p['d_r2b_b'].reshape(1, 128),
            jnp.transpose(p['d_t1_w'], (2, 3, 0, 1)).astype(_BF),
            p['d_t1_b'].reshape(1, 64),
            w36, b48)
    f = pl.pallas_call(
        _vqdec_kernel,
        grid=(4,),
        in_specs=[pl.BlockSpec((1, 56, 56, 64), lambda n: (n, 0, 0, 0)),
                  pl.BlockSpec((64, 1024), fixed),
                  pl.BlockSpec((1, 1024), fixed),
                  pl.BlockSpec((1024, 64), fixed),
                  pl.BlockSpec((1024, 64), fixed),
                  pl.BlockSpec((9, 64, 128), fixed3),
                  pl.BlockSpec((1, 128), fixed),
                  pl.BlockSpec((9, 128, 64), fixed3),
                  pl.BlockSpec((1, 64), fixed),
                  pl.BlockSpec((64, 128), fixed),
                  pl.BlockSpec((1, 128), fixed),
                  pl.BlockSpec((9, 128, 64), fixed3),
                  pl.BlockSpec((1, 64), fixed),
                  pl.BlockSpec((64, 128), fixed),
                  pl.BlockSpec((1, 128), fixed),
                  pl.BlockSpec((4, 4, 128, 64), lambda n: (0, 0, 0, 0)),
                  pl.BlockSpec((1, 64), fixed),
                  pl.BlockSpec((36, 64, 48), fixed3),
                  pl.BlockSpec((1, 48), fixed)],
        out_specs=[pl.BlockSpec((1, 56, 56, 48), lambda n: (n, 0, 0, 0)),
                   pl.BlockSpec((1, 1), fixed)],
        out_shape=[jax.ShapeDtypeStruct((4, 56, 56, 48), _F32),
                   jax.ShapeDtypeStruct((1, 1), _F32)],
        scratch_shapes=[pltpu.VMEM((58, 58, 64), _F32),
                        pltpu.VMEM((58, 58, 128), _F32),
                        pltpu.VMEM((58, 58, 256), _F32)],
    )
    return f(*args)


def kernel(x, e_c1_w, e_c1_b, e_c2_w, e_c2_b, e_c3_w, e_c3_b, e_r1a_w,
           e_r1a_b, e_r1b_w, e_r1b_b, e_r2a_w, e_r2a_b, e_r2b_w, e_r2b_b,
           q_w, q_b, embed, d_c1_w, d_c1_b, d_r1a_w, d_r1a_b, d_r1b_w,
           d_r1b_b, d_r2a_w, d_r2a_b, d_r2b_w, d_r2b_b, d_t1_w, d_t1_b,
           d_t2_w, d_t2_b):
    h = jax.nn.elu(_conv(x, e_c1_w, e_c1_b, 2, 1))
    h = jax.nn.elu(_conv(h, e_c2_w, e_c2_b, 2, 1))
    h = _conv(h, e_c3_w, e_c3_b, 1, 1)
    h = _res(h, e_r1a_w, e_r1a_b, e_r1b_w, e_r1b_b)
    h = _res(h, e_r2a_w, e_r2a_b, e_r2b_w, e_r2b_b)
    h = jax.nn.elu(h)
    z = _conv(h, q_w, q_b, 1, 0)
    z_nhwc = jnp.transpose(z, (0, 2, 3, 1))

    p = {'d_c1_w': d_c1_w, 'd_c1_b': d_c1_b,
         'd_r1a_w': d_r1a_w, 'd_r1a_b': d_r1a_b,
         'd_r1b_w': d_r1b_w, 'd_r1b_b': d_r1b_b,
         'd_r2a_w': d_r2a_w, 'd_r2a_b': d_r2a_b,
         'd_r2b_w': d_r2b_w, 'd_r2b_b': d_r2b_b,
         'd_t1_w': d_t1_w, 'd_t1_b': d_t1_b,
         'd_t2_w': d_t2_w, 'd_t2_b': d_t2_b}
    y, sq = _run_vqdec(z_nhwc, embed, p)
    diff = 0.25 * (sq[0, 0] / (4 * 56 * 56 * 64))

    dec = y.reshape(4, 56, 56, 4, 4, 3)
    dec = jnp.transpose(dec, (0, 1, 3, 2, 4, 5)).reshape(4, 224, 224, 3)
    dec = jnp.transpose(dec, (0, 3, 1, 2))
    return dec, diff


# speedup vs baseline: 1.1494x; 1.0010x over previous
import jax
import jax.numpy as jnp
from jax.experimental import pallas as pl
from jax.experimental.pallas import tpu as pltpu

_BF = jnp.bfloat16
_F32 = jnp.float32


def _conv(x, w, b, stride, pad):
    out = jax.lax.conv_general_dilated(
        x, w, (stride, stride), [(pad, pad), (pad, pad)],
        dimension_numbers=('NCHW', 'OIHW', 'NCHW'))
    return out + b[None, :, None, None]


def _res(x, w1, b1, w2, b2):
    h = _conv(jax.nn.elu(x), w1, b1, 1, 1)
    h = _conv(jax.nn.elu(h), w2, b2, 1, 0)
    return x + h


def _rtne_bf16_f32(x):
    u = jax.lax.bitcast_convert_type(x, jnp.uint32)
    t = u + jnp.uint32(0x7FFF) + ((u >> 16) & jnp.uint32(1))
    return jax.lax.bitcast_convert_type(t & jnp.uint32(0xFFFF0000), _F32)


def _elu(x):
    return jnp.where(x > 0, x, jnp.exp(x) - 1.0)


_T_TAPS = {0: ((1, 1), (0, 3)), 1: ((2, 0), (1, 2))}
_E_MAP = {-1: (1, -1), 0: (0, 0), 1: (1, 0), 2: (0, 1)}


def _vqdec_kernel(z_ref, ebf_ref, cn_ref, eh_ref, em_ref,
                  wd1_ref, bd1_ref, wr1a_ref, br1a_ref, wr1b_ref, br1b_ref,
                  wr2a_ref, br2a_ref, wr2b_ref, br2b_ref, wt1_ref, bt1_ref,
                  w36_ref, b48_ref, o_ref, sq_ref,
                  scrq_ref, scr_ref, scrt_ref):
    n = pl.program_id(0)

    z = z_ref[0].reshape(3136, 64)
    mm = jnp.dot(z.astype(_BF), ebf_ref[...], preferred_element_type=_F32)
    rown = jnp.sum(z * z, axis=1, keepdims=True)
    dist = (rown - 2.0 * mm) + cn_ref[...]
    dmin = jnp.min(dist, axis=1, keepdims=True)
    iota = jax.lax.broadcasted_iota(jnp.int32, (3136, 1024), 1)
    ind = jnp.min(jnp.where(dist == dmin, iota, jnp.int32(1 << 30)),
                  axis=1, keepdims=True)
    onehot = (iota == ind).astype(_BF)
    q = jnp.dot(onehot, eh_ref[...], preferred_element_type=_F32)
    q = q + jnp.dot(onehot, em_ref[...], preferred_element_type=_F32)
    d = q - z
    qst = z + d
    s = jnp.sum(d * d)

    @pl.when(n == 0)
    def _():
        sq_ref[...] = s.reshape(1, 1)

    @pl.when(n != 0)
    def _():
        sq_ref[...] += s.reshape(1, 1)

    scrq_ref[...] = jnp.zeros((58, 58, 64), _F32)
    scrq_ref[1:57, 1:57, :] = qst.reshape(56, 56, 64)
    scr_ref[...] = jnp.zeros((58, 58, 128), _F32)

    def taps3(src_ref, cin, w_ref):
        acc = None
        for t in range(9):
            i, j = divmod(t, 3)
            sl = src_ref[i:i + 56, j:j + 56, :].reshape(3136, cin)
            term = jnp.dot(sl.astype(_BF), w_ref[t],
                           preferred_element_type=_F32)
            acc = term if acc is None else acc + term
        return acc

    h = taps3(scrq_ref, 64, wd1_ref) + bd1_ref[...]

    def res(h, wa_ref, ba_ref, wb_ref, bb_ref):
        scr_ref[1:57, 1:57, :] = _elu(h).reshape(56, 56, 128)
        t = taps3(scr_ref, 128, wa_ref) + ba_ref[...]
        t = jnp.dot(_elu(t).astype(_BF), wb_ref[...],
                    preferred_element_type=_F32) + bb_ref[...]
        return h + t

    h = res(h, wr1a_ref, br1a_ref, wr1b_ref, br1b_ref)
    h = res(h, wr2a_ref, br2a_ref, wr2b_ref, br2b_ref)

    scr_ref[1:57, 1:57, :] = _elu(h).reshape(56, 56, 128)
    scrt_ref[...] = jnp.zeros((58, 58, 256), _F32)
    for rh in range(2):
        for rw in range(2):
            acc = None
            for (oh, kh) in _T_TAPS[rh]:
                for (ow, kw) in _T_TAPS[rw]:
                    sl = scr_ref[oh:oh + 56, ow:ow + 56, :].reshape(3136, 128)
                    term = jnp.dot(sl.astype(_BF), wt1_ref[kh, kw],
                                   preferred_element_type=_F32)
                    acc = term if acc is None else acc + term
            sub = _elu(acc + bt1_ref[...])
            c0 = (rh * 2 + rw) * 64
            scrt_ref[1:57, 1:57, c0:c0 + 64] = sub.reshape(56, 56, 64)

    acc = None
    m = 0
    for sh in range(2):
        for dh in (-1, 0, 1):
            for sw in range(2):
                for dw in (-1, 0, 1):
                    c0 = (sh * 2 + sw) * 64
                    sl = scrt_ref[1 + dh:57 + dh, 1 + dw:57 + dw,
                                  c0:c0 + 64].reshape(3136, 64)
                    term = jnp.dot(sl.astype(_BF), w36_ref[m],
                                   preferred_element_type=_F32)
                    acc = term if acc is None else acc + term
                    m += 1
    o_ref[0] = (acc + b48_ref[...]).reshape(56, 56, 48)


def _build_w36(d_t2_w):
    wt = jnp.transpose(d_t2_w, (2, 3, 0, 1))
    order = {}
    m = 0
    for sh in range(2):
        for dh in (-1, 0, 1):
            for sw in range(2):
                for dw in (-1, 0, 1):
                    order[(sh, dh, sw, dw)] = m
                    m += 1
    w36 = jnp.zeros((36, 64, 48), _F32)
    for t in range(2):
        for rh in range(2):
            ph = 2 * t + rh
            for (oh, kh) in _T_TAPS[rh]:
                sh, dh = _E_MAP[t + oh - 1]
                for u in range(2):
                    for rw in range(2):
                        pw = 2 * u + rw
                        for (ow, kw) in _T_TAPS[rw]:
                            sw, dw = _E_MAP[u + ow - 1]
                            mi = order[(sh, dh, sw, dw)]
                            c0 = (ph * 4 + pw) * 3
                            w36 = w36.at[mi, :, c0:c0 + 3].add(wt[kh, kw])
    return w36


def _run_vqdec(z_nhwc, embed, p):
    embed_bf = embed.astype(_BF)
    colnorm = (embed ** 2).sum(0, keepdims=True)
    et = jnp.transpose(embed)
    eh_f = _rtne_bf16_f32(et)
    eh, em = eh_f.astype(_BF), (et - eh_f).astype(_BF)
    w3 = lambda w: jnp.transpose(w, (2, 3, 1, 0)).reshape(9, w.shape[1], w.shape[0]).astype(_BF)
    w36 = _build_w36(p['d_t2_w']).astype(_BF)
    b48 = jnp.tile(p['d_t2_b'], 16).reshape(1, 48)
    fixed = lambda n: (0, 0)
    fixed3 = lambda n: (0, 0, 0)
    args = (z_nhwc, embed_bf, colnorm, eh, em,
            w3(p['d_c1_w']), p['d_c1_b'].reshape(1, 128),
            w3(p['d_r1a_w']), p['d_r1a_b'].reshape(1, 64),
            jnp.transpose(p['d_r1b_w'][:, :, 0, 0]).astype(_BF), p['d_r1b_b'].reshape(1, 128),
            w3(p['d_r2a_w']), p['d_r2a_b'].reshape(1, 64),
            jnp.transpose(p['d_r2b_w'][:, :, 0, 0]).astype(_BF), p['d_r2b_b'].reshape(1, 128),
            jnp.transpose(p['d_t1_w'], (2, 3, 0, 1)).astype(_BF),
            p['d_t1_b'].reshape(1, 64),
            w36, b48)
    f = pl.pallas_call(
        _vqdec_kernel,
        grid=(4,),
        in_specs=[pl.BlockSpec((1, 56, 56, 64), lambda n: (n, 0, 0, 0)),
                  pl.BlockSpec((64, 1024), fixed),
                  pl.BlockSpec((1, 1024), fixed),
                  pl.BlockSpec((1024, 64), fixed),
                  pl.BlockSpec((1024, 64), fixed),
                  pl.BlockSpec((9, 64, 128), fixed3),
                  pl.BlockSpec((1, 128), fixed),
                  pl.BlockSpec((9, 128, 64), fixed3),
                  pl.BlockSpec((1, 64), fixed),
                  pl.BlockSpec((64, 128), fixed),
                  pl.BlockSpec((1, 128), fixed),
                  pl.BlockSpec((9, 128, 64), fixed3),
                  pl.BlockSpec((1, 64), fixed),
                  pl.BlockSpec((64, 128), fixed),
                  pl.BlockSpec((1, 128), fixed),
                  pl.BlockSpec((4, 4, 128, 64), lambda n: (0, 0, 0, 0)),
                  pl.BlockSpec((1, 64), fixed),
                  pl.BlockSpec((36, 64, 48), fixed3),
                  pl.BlockSpec((1, 48), fixed)],
        out_specs=[pl.BlockSpec((1, 56, 56, 48), lambda n: (n, 0, 0, 0)),
                   pl.BlockSpec((1, 1), fixed)],
        out_shape=[jax.ShapeDtypeStruct((4, 56, 56, 48), _F32),
                   jax.ShapeDtypeStruct((1, 1), _F32)],
        scratch_shapes=[pltpu.VMEM((58, 58, 64), _F32),
                        pltpu.VMEM((58, 58, 128), _F32),
                        pltpu.VMEM((58, 58, 256), _F32)],
    )
    return f(*args)


def kernel(x, e_c1_w, e_c1_b, e_c2_w, e_c2_b, e_c3_w, e_c3_b, e_r1a_w,
           e_r1a_b, e_r1b_w, e_r1b_b, e_r2a_w, e_r2a_b, e_r2b_w, e_r2b_b,
           q_w, q_b, embed, d_c1_w, d_c1_b, d_r1a_w, d_r1a_b, d_r1b_w,
           d_r1b_b, d_r2a_w, d_r2a_b, d_r2b_w, d_r2b_b, d_t1_w, d_t1_b,
           d_t2_w, d_t2_b):
    h = jax.nn.elu(_conv(x, e_c1_w, e_c1_b, 2, 1))
    h = jax.nn.elu(_conv(h, e_c2_w, e_c2_b, 2, 1))
    h = _conv(h, e_c3_w, e_c3_b, 1, 1)
    h = _res(h, e_r1a_w, e_r1a_b, e_r1b_w, e_r1b_b)
    h = _res(h, e_r2a_w, e_r2a_b, e_r2b_w, e_r2b_b)
    h = jax.nn.elu(h)
    z = _conv(h, q_w, q_b, 1, 0)
    z_nhwc = jnp.transpose(z, (0, 2, 3, 1))

    p = {'d_c1_w': d_c1_w, 'd_c1_b': d_c1_b,
         'd_r1a_w': d_r1a_w, 'd_r1a_b': d_r1a_b,
         'd_r1b_w': d_r1b_w, 'd_r1b_b': d_r1b_b,
         'd_r2a_w': d_r2a_w, 'd_r2a_b': d_r2a_b,
         'd_r2b_w': d_r2b_w, 'd_r2b_b': d_r2b_b,
         'd_t1_w': d_t1_w, 'd_t1_b': d_t1_b,
         'd_t2_w': d_t2_w, 'd_t2_b': d_t2_b}
    y, sq = _run_vqdec(z_nhwc, embed, p)
    diff = 0.25 * (sq[0, 0] / (4 * 56 * 56 * 64))

    dec = y.reshape(4, 56, 56, 4, 4, 3)
    dec = jnp.transpose(dec, (0, 5, 1, 3, 2, 4)).reshape(4, 3, 224, 224)
    return dec, diff
